# baseline (speedup 1.0000x reference)
import dataclasses
import functools

import jax
import jax.numpy as jnp
from jax import lax
from jax.experimental import pallas as pl
from jax.experimental.pallas import tpu as pltpu
from jax.experimental.pallas import tpu_sc as plsc

NC = 2
NS = 16
NW = NC * NS
D = 32
PACK = 4
ROWW = PACK * D
BPACK = 16
CHUNK = 128
L = 16

_MESH = plsc.VectorSubcoreMesh(core_axis_name="c", subcore_axis_name="s")
_PARAMS_TILED = dataclasses.replace(
    pltpu.CompilerParams(), needs_layout_passes=False)
_PARAMS_LINEAR = dataclasses.replace(
    pltpu.CompilerParams(use_tc_tiling_on_sc=False), needs_layout_passes=False)


def _sc_dot(P4, Q4, u4, i4, um, im):
    B = u4.shape[0]
    b_per_w = B // NW
    n_ch = b_per_w // CHUNK
    n_grp = CHUNK // L

    @functools.partial(
        pl.kernel,
        mesh=_MESH,
        compiler_params=_PARAMS_TILED,
        out_type=jax.ShapeDtypeStruct((B,), jnp.float32),
        scratch_types=[
            pltpu.VMEM((b_per_w,), jnp.int32),
            pltpu.VMEM((b_per_w,), jnp.int32),
            pltpu.VMEM((b_per_w,), jnp.int32),
            pltpu.VMEM((b_per_w,), jnp.int32),
            pltpu.VMEM((b_per_w,), jnp.float32),
            pltpu.VMEM((CHUNK, ROWW), jnp.float32),
            pltpu.VMEM((CHUNK, ROWW), jnp.float32),
            pltpu.VMEM((CHUNK, ROWW), jnp.float32),
            pltpu.VMEM((CHUNK, ROWW), jnp.float32),
            pltpu.SemaphoreType.DMA,
            pltpu.SemaphoreType.DMA,
        ],
    )
    def k(P_hbm, Q_hbm, u4_hbm, i4_hbm, um_hbm, im_hbm, out_hbm,
          u4_v, i4_v, um_v, im_v, out_v, dP0, dP1, dQ0, dQ1, semA0, semA1):
        wid = lax.axis_index("s") * NC + lax.axis_index("c")
        base = wid * b_per_w
        sl_w = pl.ds(base, b_per_w)
        pltpu.sync_copy(u4_hbm.at[sl_w], u4_v)
        pltpu.sync_copy(i4_hbm.at[sl_w], i4_v)
        pltpu.sync_copy(um_hbm.at[sl_w], um_v)
        pltpu.sync_copy(im_hbm.at[sl_w], im_v)

        dP = (dP0, dP1)
        dQ = (dQ0, dQ1)
        semA = (semA0, semA1)

        def fire(c):
            sl = pl.ds(c * CHUNK, CHUNK)
            b = c % 2
            return (pltpu.async_copy(P_hbm.at[u4_v.at[sl]], dP[b], semA[b]),
                    pltpu.async_copy(Q_hbm.at[i4_v.at[sl]], dQ[b], semA[b]))

        pend = fire(0)
        for c in range(n_ch):
            nxt = fire(c + 1) if c + 1 < n_ch else None
            pend[0].wait()
            pend[1].wait()
            b = c % 2
            dPc, dQc = dP[b], dQ[b]

            @pl.loop(0, n_grp)
            def _(g):
                off = c * CHUNK + g * L
                jrow = lax.iota(jnp.int32, L) + g * L
                cbu = um_v[pl.ds(off, L)] * D
                cbi = im_v[pl.ds(off, L)] * D

                def body(d8, acc):
                    for t in range(4):
                        d = d8 * 4 + t
                        pc = plsc.load_gather(dPc, [jrow, cbu + d])
                        qc = plsc.load_gather(dQc, [jrow, cbi + d])
                        acc = acc + pc * qc
                    return acc

                out_v[pl.ds(off, L)] = lax.fori_loop(
                    0, 8, body, jnp.zeros((L,), jnp.float32))

            pend = nxt

        pltpu.sync_copy(out_v, out_hbm.at[sl_w])

    return k(P4, Q4, u4, i4, um, im)


def _sc_bias(ub2, ib2, u16, i16, um16, im16):
    B = u16.shape[0]
    b_per_w = B // NW
    n_ch = b_per_w // CHUNK
    n_grp = b_per_w // L

    @functools.partial(
        pl.kernel,
        mesh=_MESH,
        compiler_params=_PARAMS_LINEAR,
        out_type=jax.ShapeDtypeStruct((B,), jnp.float32),
        scratch_types=[
            pltpu.VMEM((b_per_w,), jnp.int32),
            pltpu.VMEM((b_per_w,), jnp.int32),
            pltpu.VMEM((b_per_w,), jnp.int32),
            pltpu.VMEM((b_per_w,), jnp.int32),
            pltpu.VMEM((b_per_w, BPACK), jnp.float32),
            pltpu.VMEM((b_per_w, BPACK), jnp.float32),
            pltpu.VMEM((b_per_w,), jnp.float32),
            pltpu.SemaphoreType.DMA,
        ],
    )
    def k(ub_hbm, ib_hbm, u16_hbm, i16_hbm, um_hbm, im_hbm, out_hbm,
          u16_v, i16_v, um_v, im_v, dbu, dbi, out_v, sem):
        wid = lax.axis_index("s") * NC + lax.axis_index("c")
        base = wid * b_per_w
        sl_w = pl.ds(base, b_per_w)
        pltpu.sync_copy(u16_hbm.at[sl_w], u16_v)
        pltpu.sync_copy(i16_hbm.at[sl_w], i16_v)
        pltpu.sync_copy(um_hbm.at[sl_w], um_v)
        pltpu.sync_copy(im_hbm.at[sl_w], im_v)
        gathers = []
        for c in range(n_ch):
            sl = pl.ds(c * CHUNK, CHUNK)
            gathers.append(pltpu.async_copy(ub_hbm.at[u16_v.at[sl]], dbu.at[sl], sem))
            gathers.append(pltpu.async_copy(ib_hbm.at[i16_v.at[sl]], dbi.at[sl], sem))
        for g in gathers:
            g.wait()

        @pl.loop(0, n_grp)
        def _(g):
            off = g * L
            jrow = lax.iota(jnp.int32, L) + off
            bu = plsc.load_gather(dbu, [jrow, um_v[pl.ds(off, L)]])
            bi = plsc.load_gather(dbi, [jrow, im_v[pl.ds(off, L)]])
            out_v[pl.ds(off, L)] = bu + bi

        pltpu.sync_copy(out_v, out_hbm.at[sl_w])

    return k(ub2, ib2, u16, i16, um16, im16)


def _add_body(a_ref, b_ref, o_ref):
    o_ref[...] = a_ref[...] + b_ref[...]


def _tc_add(a, b):
    return pl.pallas_call(
        _add_body,
        out_shape=jax.ShapeDtypeStruct(a.shape, a.dtype),
    )(a, b)


def kernel(user_id, item_id, P, Q, user_bias, item_bias):
    P4 = P.reshape(P.shape[0] // PACK, ROWW)
    Q4 = Q.reshape(Q.shape[0] // PACK, ROWW)
    ub2 = user_bias.reshape(user_bias.shape[0] // BPACK, BPACK)
    ib2 = item_bias.reshape(item_bias.shape[0] // BPACK, BPACK)
    dot = _sc_dot(P4, Q4, user_id >> 2, item_id >> 2,
                  user_id & 3, item_id & 3)
    bsum = _sc_bias(ub2, ib2, user_id >> 4, item_id >> 4,
                    user_id & 15, item_id & 15)
    return _tc_add(dot, bsum)


# speedup vs baseline: 1.0018x; 1.0018x over previous
import dataclasses
import functools

import jax
import jax.numpy as jnp
from jax import lax
from jax.experimental import pallas as pl
from jax.experimental.pallas import tpu as pltpu
from jax.experimental.pallas import tpu_sc as plsc

NC = 2
NS = 16
NW = NC * NS
D = 32
PACK = 4
ROWW = PACK * D
CHUNK = 128
L = 16

_MESH = plsc.VectorSubcoreMesh(core_axis_name="c", subcore_axis_name="s")
_PARAMS = dataclasses.replace(
    pltpu.CompilerParams(), needs_layout_passes=False)


def _sc_mf(P4, Q4, ub, ib, u4, i4, um, im, uid, iid):
    B = uid.shape[0]
    b_per_w = B // NW
    n_ch = b_per_w // CHUNK
    n_grp = CHUNK // L

    @functools.partial(
        pl.kernel,
        mesh=_MESH,
        compiler_params=_PARAMS,
        out_type=jax.ShapeDtypeStruct((B,), jnp.float32),
        scratch_types=[
            pltpu.VMEM((b_per_w,), jnp.int32),
            pltpu.VMEM((b_per_w,), jnp.int32),
            pltpu.VMEM((b_per_w,), jnp.int32),
            pltpu.VMEM((b_per_w,), jnp.int32),
            pltpu.VMEM((b_per_w,), jnp.int32),
            pltpu.VMEM((b_per_w,), jnp.int32),
            pltpu.VMEM((b_per_w,), jnp.float32),
            pltpu.VMEM((b_per_w,), jnp.float32),
            pltpu.VMEM((b_per_w,), jnp.float32),
            pltpu.VMEM((CHUNK, ROWW), jnp.float32),
            pltpu.VMEM((CHUNK, ROWW), jnp.float32),
            pltpu.VMEM((CHUNK, ROWW), jnp.float32),
            pltpu.VMEM((CHUNK, ROWW), jnp.float32),
            pltpu.SemaphoreType.DMA,
            pltpu.SemaphoreType.DMA,
            pltpu.SemaphoreType.DMA,
        ],
    )
    def k(P_hbm, Q_hbm, ub_hbm, ib_hbm, u4_hbm, i4_hbm, um_hbm, im_hbm,
          uid_hbm, iid_hbm, out_hbm,
          u4_v, i4_v, um_v, im_v, uid_v, iid_v, bu_v, bi_v, out_v,
          dP0, dP1, dQ0, dQ1, semA0, semA1, semB):
        wid = lax.axis_index("s") * NC + lax.axis_index("c")
        base = wid * b_per_w
        sl_w = pl.ds(base, b_per_w)
        pltpu.sync_copy(u4_hbm.at[sl_w], u4_v)
        pltpu.sync_copy(i4_hbm.at[sl_w], i4_v)
        pltpu.sync_copy(um_hbm.at[sl_w], um_v)
        pltpu.sync_copy(im_hbm.at[sl_w], im_v)
        pltpu.sync_copy(uid_hbm.at[sl_w], uid_v)
        pltpu.sync_copy(iid_hbm.at[sl_w], iid_v)

        dP = (dP0, dP1)
        dQ = (dQ0, dQ1)
        semA = (semA0, semA1)

        bias_copies = []
        for c in range(n_ch):
            sl = pl.ds(c * CHUNK, CHUNK)
            bias_copies.append(
                pltpu.async_copy(ub_hbm.at[uid_v.at[sl]], bu_v.at[sl], semB))
            bias_copies.append(
                pltpu.async_copy(ib_hbm.at[iid_v.at[sl]], bi_v.at[sl], semB))

        def fire(c):
            sl = pl.ds(c * CHUNK, CHUNK)
            b = c % 2
            return (pltpu.async_copy(P_hbm.at[u4_v.at[sl]], dP[b], semA[b]),
                    pltpu.async_copy(Q_hbm.at[i4_v.at[sl]], dQ[b], semA[b]))

        pend = fire(0)
        for bc in bias_copies:
            bc.wait()

        for c in range(n_ch):
            nxt = fire(c + 1) if c + 1 < n_ch else None
            pend[0].wait()
            pend[1].wait()
            b = c % 2
            dPc, dQc = dP[b], dQ[b]

            @pl.loop(0, n_grp)
            def _(g):
                off = c * CHUNK + g * L
                jrow = lax.iota(jnp.int32, L) + g * L
                cbu = um_v[pl.ds(off, L)] * D
                cbi = im_v[pl.ds(off, L)] * D
                acc0 = bu_v[pl.ds(off, L)] + bi_v[pl.ds(off, L)]

                def body(d8, acc):
                    for t in range(4):
                        d = d8 * 4 + t
                        pc = plsc.load_gather(dPc, [jrow, cbu + d])
                        qc = plsc.load_gather(dQc, [jrow, cbi + d])
                        acc = acc + pc * qc
                    return acc

                out_v[pl.ds(off, L)] = lax.fori_loop(0, 8, body, acc0)

            pend = nxt

        pltpu.sync_copy(out_v, out_hbm.at[sl_w])

    return k(P4, Q4, ub, ib, u4, i4, um, im, uid, iid)


def kernel(user_id, item_id, P, Q, user_bias, item_bias):
    P4 = P.reshape(P.shape[0] // PACK, ROWW)
    Q4 = Q.reshape(Q.shape[0] // PACK, ROWW)
    ub = user_bias.reshape(-1)
    ib = item_bias.reshape(-1)
    u4 = user_id >> 2
    i4 = item_id >> 2
    um = user_id & 3
    im = item_id & 3
    return _sc_mf(P4, Q4, ub, ib, u4, i4, um, im, user_id, item_id)
